# baseline (speedup 1.0000x reference)
import functools

import jax
import jax.numpy as jnp
from jax import lax
from jax.experimental import pallas as pl
from jax.experimental.pallas import tpu as pltpu
from jax.experimental.pallas import tpu_sc as plsc

B = 4096
N = 200
D = 128
S = 50
NUM_REV = 5

_F = 16
_NC = 2
_NS = 16
_NW = _NC * _NS

_MESH = dict(core_axis_name="c", subcore_axis_name="s",
             num_cores=_NC, num_subcores=_NS)


def _wid():
    return lax.axis_index("s") * _NC + lax.axis_index("c")


def _gather_rows(emb_hbm, node_hbm, rows_hbm, node_v, emb_idx_v, rows_v,
                 sem_n, sem_a, base, lbase, bpw, lane):
    cp_n = pltpu.async_copy(node_hbm.at[pl.ds(base, bpw)], node_v, sem_n)
    cp_n.wait()
    for i in range(bpw // 16):
        sl = pl.ds(i * 16, 16)
        emb_idx_v[sl] = (base + i * 16 + lane) * N + node_v[sl]
    return pltpu.async_copy(emb_hbm.at[emb_idx_v], rows_v, sem_a)


@functools.lru_cache(maxsize=4)
def _make_sc_main(bg):
    bpw_g = bg // _NW
    bpw_f = B // _NW

    @functools.partial(
        pl.kernel,
        mesh=plsc.VectorSubcoreMesh(**_MESH),
        out_type=(
            jax.ShapeDtypeStruct((bg, D), jnp.float32),
            jax.ShapeDtypeStruct((_F, B), jnp.float32),
        ),
        scratch_types=[
            pltpu.VMEM((bpw_g,), jnp.int32),
            pltpu.VMEM((bpw_f,), jnp.int32),
            pltpu.VMEM((bpw_f,), jnp.float32),
            pltpu.VMEM((2, bpw_f), jnp.float32),
            pltpu.VMEM((bpw_g,), jnp.int32),
            pltpu.VMEM((bpw_f,), jnp.int32),
            pltpu.VMEM((bpw_f,), jnp.int32),
            pltpu.VMEM((bpw_g, D), jnp.float32),
            pltpu.VMEM((_F, bpw_f), jnp.float32),
            pltpu.SemaphoreType.DMA,
            pltpu.SemaphoreType.DMA,
            pltpu.SemaphoreType.DMA,
            pltpu.SemaphoreType.DMA,
            pltpu.SemaphoreType.DMA,
            pltpu.SemaphoreType.DMA,
        ],
    )
    def sc_main(emb_hbm, node_hbm, code_hbm, ct_hbm, twt_hbm, stackt_hbm,
                rows_hbm, featst_hbm, node_v, code_v, ct_v, tw_v, emb_idx_v,
                rc_idx_v, rc_v, rows_v, featst_v,
                sem_a, sem_b, sem_n, sem_c, sem_t, sem_w):
        wid = _wid()
        fbase = wid * bpw_f
        lane = lax.broadcasted_iota(jnp.int32, (16,), 0)

        cp_c = pltpu.async_copy(code_hbm.at[pl.ds(fbase, bpw_f)], code_v, sem_c)
        cp_t = pltpu.async_copy(ct_hbm.at[pl.ds(fbase, bpw_f)], ct_v, sem_t)
        cp_w = pltpu.async_copy(
            twt_hbm.at[0, :, pl.ds(fbase, bpw_f)], tw_v, sem_w)

        emb_cp = _gather_rows(emb_hbm, node_hbm, rows_hbm, node_v, emb_idx_v,
                              rows_v, sem_n, sem_a, wid * bpw_g, wid * bpw_g,
                              bpw_g, lane)

        cp_c.wait()
        for i in range(bpw_f // 16):
            sl = pl.ds(i * 16, 16)
            rc_idx_v[sl] = (code_v[sl] >> 2) * B + fbase + i * 16 + lane
        rc_cp = pltpu.async_copy(stackt_hbm.at[rc_idx_v], rc_v, sem_b)

        zero16 = jnp.zeros((16,), jnp.float32)
        for c in range(10, _F):
            for i in range(bpw_f // 16):
                featst_v[c, pl.ds(i * 16, 16)] = zero16

        cp_t.wait()
        cp_w.wait()
        for i in range(bpw_f // 16):
            sl = pl.ds(i * 16, 16)
            featst_v[0, sl] = ct_v[sl] / tw_v[1, sl]

        rc_cp.wait()
        for i in range(bpw_f // 16):
            sl = pl.ds(i * 16, 16)
            code16 = code_v[sl]
            rc16 = jnp.clip(rc_v[sl], 0, NUM_REV - 1)
            for c in range(NUM_REV):
                featst_v[1 + c, sl] = jnp.where(
                    rc16 == c, 1.0, 0.0).astype(jnp.float32)
            btf = (code16 & 1).astype(jnp.float32)
            inff = ((code16 >> 1) & 1).astype(jnp.float32)
            featst_v[6, sl] = 1.0 - btf
            featst_v[7, sl] = btf
            featst_v[8, sl] = 1.0 - inff
            featst_v[9, sl] = inff
        pltpu.sync_copy(featst_v, featst_hbm.at[:, pl.ds(fbase, bpw_f)])

        emb_cp.wait()
        pltpu.sync_copy(rows_v, rows_hbm.at[pl.ds(wid * bpw_g, bpw_g)])

    return sc_main


@functools.lru_cache(maxsize=4)
def _make_sc_tail(bc, off):
    bpw = bc // _NW

    @functools.partial(
        pl.kernel,
        mesh=plsc.VectorSubcoreMesh(**_MESH),
        out_type=jax.ShapeDtypeStruct((bc, D), jnp.float32),
        scratch_types=[
            pltpu.VMEM((bpw,), jnp.int32),
            pltpu.VMEM((bpw,), jnp.int32),
            pltpu.VMEM((bpw, D), jnp.float32),
            pltpu.SemaphoreType.DMA,
            pltpu.SemaphoreType.DMA,
        ],
    )
    def sc_tail(emb_hbm, node_hbm, rows_hbm, node_v, emb_idx_v, rows_v,
                sem_n, sem_a):
        wid = _wid()
        lane = lax.broadcasted_iota(jnp.int32, (16,), 0)
        cp = _gather_rows(emb_hbm, node_hbm, rows_hbm, node_v, emb_idx_v,
                          rows_v, sem_n, sem_a, off + wid * bpw, wid * bpw,
                          bpw, lane)
        cp.wait()
        pltpu.sync_copy(rows_v, rows_hbm.at[pl.ds(wid * bpw, bpw)])

    return sc_tail


_BLK = 1024


def _tc_body(x_ref, ft_ref, w_ref, b_ref, out_ref):
    w0 = w_ref[:D, :]
    w1 = w_ref[D:, :]
    ft = ft_ref[:10, :]
    out_ref[...] = (
        jnp.dot(x_ref[...], w0, preferred_element_type=jnp.float32)
        + lax.dot_general(ft, w1, (((0,), (0,)), ((), ())),
                          preferred_element_type=jnp.float32)
        + b_ref[...][None, :]
    )


def _tc_body_alias(x_ref, ft_ref, w_ref, b_ref, prev_ref, out_ref):
    _tc_body(x_ref, ft_ref, w_ref, b_ref, out_ref)


@functools.lru_cache(maxsize=4)
def _make_tc_call(bc, off):
    blk_off = off // _BLK
    in_specs = [
        pl.BlockSpec((_BLK, D), lambda i: (i, 0)),
        pl.BlockSpec((_F, _BLK), lambda i: (0, i + blk_off)),
        pl.BlockSpec((D + 10, D), lambda i: (0, 0)),
        pl.BlockSpec((D,), lambda i: (0,)),
    ]
    if off == 0:
        return pl.pallas_call(
            _tc_body,
            grid=(bc // _BLK,),
            in_specs=in_specs,
            out_specs=pl.BlockSpec((_BLK, D), lambda i: (i + blk_off, 0)),
            out_shape=jax.ShapeDtypeStruct((B, D), jnp.float32),
        )
    return pl.pallas_call(
        _tc_body_alias,
        grid=(bc // _BLK,),
        in_specs=in_specs + [pl.BlockSpec(memory_space=pltpu.MemorySpace.HBM)],
        out_specs=pl.BlockSpec((_BLK, D), lambda i: (i + blk_off, 0)),
        out_shape=jax.ShapeDtypeStruct((B, D), jnp.float32),
        input_output_aliases={4: 0},
    )


_NCHUNK = 2


def kernel(embeddings, current_node, revisit_count_stack, step_idx,
           backtrack_budget_reached, confirmed_infeasible,
           current_time, time_windows, W, b):
    emb2d = embeddings.reshape(B * N, D)
    code = (step_idx.astype(jnp.int32) * 4
            + backtrack_budget_reached.astype(jnp.int32)
            + 2 * confirmed_infeasible.astype(jnp.int32))
    stackt = revisit_count_stack.T.reshape(S * B).astype(jnp.int32)
    twt = time_windows.transpose(1, 2, 0)
    node = current_node.astype(jnp.int32)

    bc = B // _NCHUNK
    rows0, featst = _make_sc_main(bc)(
        emb2d, node, code, current_time, twt, stackt)
    out = _make_tc_call(bc, 0)(rows0, featst, W, b)
    for ci in range(1, _NCHUNK):
        rows = _make_sc_tail(bc, ci * bc)(emb2d, node)
        out = _make_tc_call(bc, ci * bc)(rows, featst, W, b, out)
    return out


# speedup vs baseline: 1.1026x; 1.1026x over previous
import functools

import jax
import jax.numpy as jnp
from jax import lax
from jax.experimental import pallas as pl
from jax.experimental.pallas import tpu as pltpu
from jax.experimental.pallas import tpu_sc as plsc

B = 4096
N = 200
D = 128
S = 50
NUM_REV = 5

_F = 16
_NC = 2
_NS = 16
_NW = _NC * _NS

_MESH = dict(core_axis_name="c", subcore_axis_name="s",
             num_cores=_NC, num_subcores=_NS)


def _wid():
    return lax.axis_index("s") * _NC + lax.axis_index("c")


def _gather_rows(emb_hbm, node_hbm, rows_hbm, node_v, emb_idx_v, rows_v,
                 sem_n, sem_a, base, lbase, bpw, lane):
    cp_n = pltpu.async_copy(node_hbm.at[pl.ds(base, bpw)], node_v, sem_n)
    cp_n.wait()
    for i in range(bpw // 16):
        sl = pl.ds(i * 16, 16)
        emb_idx_v[sl] = (base + i * 16 + lane) * N + node_v[sl]
    return pltpu.async_copy(emb_hbm.at[emb_idx_v], rows_v, sem_a)


@functools.lru_cache(maxsize=4)
def _make_sc_main(bg):
    bpw_g = bg // _NW
    bpw_f = B // _NW

    @functools.partial(
        pl.kernel,
        mesh=plsc.VectorSubcoreMesh(**_MESH),
        out_type=(
            jax.ShapeDtypeStruct((bg, D), jnp.float32),
            jax.ShapeDtypeStruct((_F, B), jnp.float32),
        ),
        scratch_types=[
            pltpu.VMEM((bpw_g,), jnp.int32),
            pltpu.VMEM((bpw_f,), jnp.int32),
            pltpu.VMEM((bpw_f,), jnp.float32),
            pltpu.VMEM((2, bpw_f), jnp.float32),
            pltpu.VMEM((bpw_g,), jnp.int32),
            pltpu.VMEM((bpw_f,), jnp.int32),
            pltpu.VMEM((bpw_f,), jnp.int32),
            pltpu.VMEM((bpw_g, D), jnp.float32),
            pltpu.VMEM((_F, bpw_f), jnp.float32),
            pltpu.SemaphoreType.DMA,
            pltpu.SemaphoreType.DMA,
            pltpu.SemaphoreType.DMA,
            pltpu.SemaphoreType.DMA,
            pltpu.SemaphoreType.DMA,
            pltpu.SemaphoreType.DMA,
        ],
    )
    def sc_main(emb_hbm, node_hbm, code_hbm, ct_hbm, twt_hbm, stackt_hbm,
                rows_hbm, featst_hbm, node_v, code_v, ct_v, tw_v, emb_idx_v,
                rc_idx_v, rc_v, rows_v, featst_v,
                sem_a, sem_b, sem_n, sem_c, sem_t, sem_w):
        wid = _wid()
        fbase = wid * bpw_f
        lane = lax.broadcasted_iota(jnp.int32, (16,), 0)

        cp_c = pltpu.async_copy(code_hbm.at[pl.ds(fbase, bpw_f)], code_v, sem_c)
        cp_t = pltpu.async_copy(ct_hbm.at[pl.ds(fbase, bpw_f)], ct_v, sem_t)
        cp_w = pltpu.async_copy(
            twt_hbm.at[0, :, pl.ds(fbase, bpw_f)], tw_v, sem_w)

        emb_cp = _gather_rows(emb_hbm, node_hbm, rows_hbm, node_v, emb_idx_v,
                              rows_v, sem_n, sem_a, wid * bpw_g, wid * bpw_g,
                              bpw_g, lane)

        cp_c.wait()
        for i in range(bpw_f // 16):
            sl = pl.ds(i * 16, 16)
            rc_idx_v[sl] = (code_v[sl] >> 2) * B + fbase + i * 16 + lane
        rc_cp = pltpu.async_copy(stackt_hbm.at[rc_idx_v], rc_v, sem_b)

        zero16 = jnp.zeros((16,), jnp.float32)
        for c in range(10, _F):
            for i in range(bpw_f // 16):
                featst_v[c, pl.ds(i * 16, 16)] = zero16

        cp_t.wait()
        cp_w.wait()
        for i in range(bpw_f // 16):
            sl = pl.ds(i * 16, 16)
            featst_v[0, sl] = ct_v[sl] / tw_v[1, sl]

        rc_cp.wait()
        for i in range(bpw_f // 16):
            sl = pl.ds(i * 16, 16)
            code16 = code_v[sl]
            rc16 = jnp.clip(rc_v[sl], 0, NUM_REV - 1)
            for c in range(NUM_REV):
                featst_v[1 + c, sl] = jnp.where(
                    rc16 == c, 1.0, 0.0).astype(jnp.float32)
            btf = (code16 & 1).astype(jnp.float32)
            inff = ((code16 >> 1) & 1).astype(jnp.float32)
            featst_v[6, sl] = 1.0 - btf
            featst_v[7, sl] = btf
            featst_v[8, sl] = 1.0 - inff
            featst_v[9, sl] = inff
        pltpu.sync_copy(featst_v, featst_hbm.at[:, pl.ds(fbase, bpw_f)])

        emb_cp.wait()
        pltpu.sync_copy(rows_v, rows_hbm.at[pl.ds(wid * bpw_g, bpw_g)])

    return sc_main


@functools.lru_cache(maxsize=4)
def _make_sc_tail(bc, off):
    bpw = bc // _NW

    @functools.partial(
        pl.kernel,
        mesh=plsc.VectorSubcoreMesh(**_MESH),
        out_type=jax.ShapeDtypeStruct((bc, D), jnp.float32),
        scratch_types=[
            pltpu.VMEM((bpw,), jnp.int32),
            pltpu.VMEM((bpw,), jnp.int32),
            pltpu.VMEM((bpw, D), jnp.float32),
            pltpu.SemaphoreType.DMA,
            pltpu.SemaphoreType.DMA,
        ],
    )
    def sc_tail(emb_hbm, node_hbm, rows_hbm, node_v, emb_idx_v, rows_v,
                sem_n, sem_a):
        wid = _wid()
        lane = lax.broadcasted_iota(jnp.int32, (16,), 0)
        cp = _gather_rows(emb_hbm, node_hbm, rows_hbm, node_v, emb_idx_v,
                          rows_v, sem_n, sem_a, off + wid * bpw, wid * bpw,
                          bpw, lane)
        cp.wait()
        pltpu.sync_copy(rows_v, rows_hbm.at[pl.ds(wid * bpw, bpw)])

    return sc_tail


_BLK = 1024


def _tc_body(x_ref, ft_ref, w_ref, b_ref, out_ref):
    w0 = w_ref[:D, :]
    w1 = w_ref[D:, :]
    ft = ft_ref[:10, :]
    out_ref[...] = (
        jnp.dot(x_ref[...], w0, preferred_element_type=jnp.float32)
        + lax.dot_general(ft, w1, (((0,), (0,)), ((), ())),
                          preferred_element_type=jnp.float32)
        + b_ref[...][None, :]
    )


def _tc_body_alias(x_ref, ft_ref, w_ref, b_ref, prev_ref, out_ref):
    _tc_body(x_ref, ft_ref, w_ref, b_ref, out_ref)


@functools.lru_cache(maxsize=4)
def _make_tc_call(bc, off):
    blk_off = off // _BLK
    in_specs = [
        pl.BlockSpec((_BLK, D), lambda i: (i, 0)),
        pl.BlockSpec((_F, _BLK), lambda i: (0, i + blk_off)),
        pl.BlockSpec((D + 10, D), lambda i: (0, 0)),
        pl.BlockSpec((D,), lambda i: (0,)),
    ]
    if off == 0:
        return pl.pallas_call(
            _tc_body,
            grid=(bc // _BLK,),
            in_specs=in_specs,
            out_specs=pl.BlockSpec((_BLK, D), lambda i: (i + blk_off, 0)),
            out_shape=jax.ShapeDtypeStruct((B, D), jnp.float32),
        )
    return pl.pallas_call(
        _tc_body_alias,
        grid=(bc // _BLK,),
        in_specs=in_specs + [pl.BlockSpec(memory_space=pltpu.MemorySpace.HBM)],
        out_specs=pl.BlockSpec((_BLK, D), lambda i: (i + blk_off, 0)),
        out_shape=jax.ShapeDtypeStruct((B, D), jnp.float32),
        input_output_aliases={4: 0},
    )


_NCHUNK = 1


def kernel(embeddings, current_node, revisit_count_stack, step_idx,
           backtrack_budget_reached, confirmed_infeasible,
           current_time, time_windows, W, b):
    emb2d = embeddings.reshape(B * N, D)
    code = (step_idx.astype(jnp.int32) * 4
            + backtrack_budget_reached.astype(jnp.int32)
            + 2 * confirmed_infeasible.astype(jnp.int32))
    stackt = revisit_count_stack.T.reshape(S * B).astype(jnp.int32)
    twt = time_windows.transpose(1, 2, 0)
    node = current_node.astype(jnp.int32)

    bc = B // _NCHUNK
    rows0, featst = _make_sc_main(bc)(
        emb2d, node, code, current_time, twt, stackt)
    out = _make_tc_call(bc, 0)(rows0, featst, W, b)
    for ci in range(1, _NCHUNK):
        rows = _make_sc_tail(bc, ci * bc)(emb2d, node)
        out = _make_tc_call(bc, ci * bc)(rows, featst, W, b, out)
    return out


# speedup vs baseline: 1.1445x; 1.0380x over previous
import functools

import jax
import jax.numpy as jnp
from jax import lax
from jax.experimental import pallas as pl
from jax.experimental.pallas import tpu as pltpu
from jax.experimental.pallas import tpu_sc as plsc

B = 4096
N = 200
D = 128
S = 50
NUM_REV = 5

_F = 16
_NC = 2
_NS = 16
_NW = _NC * _NS

_MESH = dict(core_axis_name="c", subcore_axis_name="s",
             num_cores=_NC, num_subcores=_NS)


def _wid():
    return lax.axis_index("s") * _NC + lax.axis_index("c")


def _gather_rows(emb_hbm, node_hbm, rows_hbm, node_v, emb_idx_v, rows_v,
                 sem_n, sem_a, base, lbase, bpw, lane):
    cp_n = pltpu.async_copy(node_hbm.at[pl.ds(base, bpw)], node_v, sem_n)
    cp_n.wait()
    for i in range(bpw // 16):
        sl = pl.ds(i * 16, 16)
        emb_idx_v[sl] = (base + i * 16 + lane) * N + node_v[sl]
    return pltpu.async_copy(emb_hbm.at[emb_idx_v], rows_v, sem_a)


@functools.lru_cache(maxsize=4)
def _make_sc_main(bg):
    bpw_g = bg // _NW
    bpw_f = B // _NW

    @functools.partial(
        pl.kernel,
        mesh=plsc.VectorSubcoreMesh(**_MESH),
        out_type=(
            jax.ShapeDtypeStruct((bg, D), jnp.float32),
            jax.ShapeDtypeStruct((_F, B), jnp.float32),
        ),
        scratch_types=[
            pltpu.VMEM((bpw_g,), jnp.int32),
            pltpu.VMEM((bpw_f,), jnp.int32),
            pltpu.VMEM((bpw_f,), jnp.float32),
            pltpu.VMEM((2, bpw_f), jnp.float32),
            pltpu.VMEM((bpw_g,), jnp.int32),
            pltpu.VMEM((bpw_g, D), jnp.float32),
            pltpu.VMEM((_F, bpw_f), jnp.float32),
            pltpu.SemaphoreType.DMA,
            pltpu.SemaphoreType.DMA,
            pltpu.SemaphoreType.DMA,
            pltpu.SemaphoreType.DMA,
            pltpu.SemaphoreType.DMA,
        ],
    )
    def sc_main(emb_hbm, node_hbm, code_hbm, ct_hbm, twt_hbm,
                rows_hbm, featst_hbm, node_v, code_v, ct_v, tw_v, emb_idx_v,
                rows_v, featst_v,
                sem_a, sem_n, sem_c, sem_t, sem_w):
        wid = _wid()
        fbase = wid * bpw_f
        lane = lax.broadcasted_iota(jnp.int32, (16,), 0)

        cp_c = pltpu.async_copy(code_hbm.at[pl.ds(fbase, bpw_f)], code_v, sem_c)
        cp_t = pltpu.async_copy(ct_hbm.at[pl.ds(fbase, bpw_f)], ct_v, sem_t)
        cp_w = pltpu.async_copy(
            twt_hbm.at[0, :, pl.ds(fbase, bpw_f)], tw_v, sem_w)

        emb_cp = _gather_rows(emb_hbm, node_hbm, rows_hbm, node_v, emb_idx_v,
                              rows_v, sem_n, sem_a, wid * bpw_g, wid * bpw_g,
                              bpw_g, lane)

        cp_t.wait()
        cp_w.wait()
        for i in range(bpw_f // 16):
            sl = pl.ds(i * 16, 16)
            featst_v[0, sl] = ct_v[sl] / tw_v[1, sl]

        cp_c.wait()
        for i in range(bpw_f // 16):
            sl = pl.ds(i * 16, 16)
            code16 = code_v[sl]
            btf = (code16 & 1).astype(jnp.float32)
            inff = ((code16 >> 1) & 1).astype(jnp.float32)
            featst_v[6, sl] = 1.0 - btf
            featst_v[7, sl] = btf
            featst_v[8, sl] = 1.0 - inff
            featst_v[9, sl] = inff
            featst_v[10, sl] = (code16 >> 2).astype(jnp.float32)
        pltpu.sync_copy(featst_v, featst_hbm.at[:, pl.ds(fbase, bpw_f)])

        emb_cp.wait()
        pltpu.sync_copy(rows_v, rows_hbm.at[pl.ds(wid * bpw_g, bpw_g)])

    return sc_main


@functools.lru_cache(maxsize=4)
def _make_sc_tail(bc, off):
    bpw = bc // _NW

    @functools.partial(
        pl.kernel,
        mesh=plsc.VectorSubcoreMesh(**_MESH),
        out_type=jax.ShapeDtypeStruct((bc, D), jnp.float32),
        scratch_types=[
            pltpu.VMEM((bpw,), jnp.int32),
            pltpu.VMEM((bpw,), jnp.int32),
            pltpu.VMEM((bpw, D), jnp.float32),
            pltpu.SemaphoreType.DMA,
            pltpu.SemaphoreType.DMA,
        ],
    )
    def sc_tail(emb_hbm, node_hbm, rows_hbm, node_v, emb_idx_v, rows_v,
                sem_n, sem_a):
        wid = _wid()
        lane = lax.broadcasted_iota(jnp.int32, (16,), 0)
        cp = _gather_rows(emb_hbm, node_hbm, rows_hbm, node_v, emb_idx_v,
                          rows_v, sem_n, sem_a, off + wid * bpw, wid * bpw,
                          bpw, lane)
        cp.wait()
        pltpu.sync_copy(rows_v, rows_hbm.at[pl.ds(wid * bpw, bpw)])

    return sc_tail


_BLK = 1024


def _tc_body(x_ref, ft_ref, st_ref, w_ref, b_ref, out_ref):
    blk = x_ref.shape[0]
    w0 = w_ref[:D, :]
    w1 = w_ref[D:, :]
    sidx = ft_ref[10:11, :].astype(jnp.int32)
    srow = lax.broadcasted_iota(jnp.int32, (S, blk), 0)
    rc = jnp.sum(jnp.where(srow == sidx, st_ref[...], 0),
                 axis=0, keepdims=True)
    rc = jnp.clip(rc, 0, NUM_REV - 1)
    crow = lax.broadcasted_iota(jnp.int32, (NUM_REV, blk), 0)
    oh5 = jnp.where(crow == rc, 1.0, 0.0)
    ft = jnp.concatenate(
        [ft_ref[0:1, :], oh5, ft_ref[6:10, :]], axis=0)
    out_ref[...] = (
        jnp.dot(x_ref[...], w0, preferred_element_type=jnp.float32)
        + lax.dot_general(ft, w1, (((0,), (0,)), ((), ())),
                          preferred_element_type=jnp.float32)
        + b_ref[...][None, :]
    )


def _tc_body_alias(x_ref, ft_ref, st_ref, w_ref, b_ref, prev_ref, out_ref):
    _tc_body(x_ref, ft_ref, st_ref, w_ref, b_ref, out_ref)


@functools.lru_cache(maxsize=4)
def _make_tc_call(bc, off):
    blk_off = off // _BLK
    in_specs = [
        pl.BlockSpec((_BLK, D), lambda i: (i, 0)),
        pl.BlockSpec((_F, _BLK), lambda i: (0, i + blk_off)),
        pl.BlockSpec((S, _BLK), lambda i: (0, i + blk_off)),
        pl.BlockSpec((D + 10, D), lambda i: (0, 0)),
        pl.BlockSpec((D,), lambda i: (0,)),
    ]
    if off == 0:
        return pl.pallas_call(
            _tc_body,
            grid=(bc // _BLK,),
            in_specs=in_specs,
            out_specs=pl.BlockSpec((_BLK, D), lambda i: (i + blk_off, 0)),
            out_shape=jax.ShapeDtypeStruct((B, D), jnp.float32),
        )
    return pl.pallas_call(
        _tc_body_alias,
        grid=(bc // _BLK,),
        in_specs=in_specs + [pl.BlockSpec(memory_space=pltpu.MemorySpace.HBM)],
        out_specs=pl.BlockSpec((_BLK, D), lambda i: (i + blk_off, 0)),
        out_shape=jax.ShapeDtypeStruct((B, D), jnp.float32),
        input_output_aliases={5: 0},
    )


_NCHUNK = 1


def kernel(embeddings, current_node, revisit_count_stack, step_idx,
           backtrack_budget_reached, confirmed_infeasible,
           current_time, time_windows, W, b):
    emb2d = embeddings.reshape(B * N, D)
    code = (step_idx.astype(jnp.int32) * 4
            + backtrack_budget_reached.astype(jnp.int32)
            + 2 * confirmed_infeasible.astype(jnp.int32))
    stackt = revisit_count_stack.T.astype(jnp.int32)
    twt = time_windows.transpose(1, 2, 0)
    node = current_node.astype(jnp.int32)

    bc = B // _NCHUNK
    rows0, featst = _make_sc_main(bc)(
        emb2d, node, code, current_time, twt)
    out = _make_tc_call(bc, 0)(rows0, featst, stackt, W, b)
    for ci in range(1, _NCHUNK):
        rows = _make_sc_tail(bc, ci * bc)(emb2d, node)
        out = _make_tc_call(bc, ci * bc)(rows, featst, stackt, W, b, out)
    return out


# speedup vs baseline: 1.1979x; 1.0467x over previous
import functools

import jax
import jax.numpy as jnp
from jax import lax
from jax.experimental import pallas as pl
from jax.experimental.pallas import tpu as pltpu
from jax.experimental.pallas import tpu_sc as plsc

B = 4096
N = 200
D = 128
S = 50
NUM_REV = 5

_F = 16
_NC = 2
_NS = 16
_NW = _NC * _NS

_MESH = dict(core_axis_name="c", subcore_axis_name="s",
             num_cores=_NC, num_subcores=_NS)


def _wid():
    return lax.axis_index("s") * _NC + lax.axis_index("c")


def _gather_rows(emb_hbm, node_hbm, rows_hbm, node_v, emb_idx_v, rows_v,
                 sem_n, sem_a, base, lbase, bpw, lane):
    cp_n = pltpu.async_copy(node_hbm.at[pl.ds(base, bpw)], node_v, sem_n)
    cp_n.wait()
    for i in range(bpw // 16):
        sl = pl.ds(i * 16, 16)
        emb_idx_v[sl] = (base + i * 16 + lane) * N + node_v[sl]
    return pltpu.async_copy(emb_hbm.at[emb_idx_v], rows_v, sem_a)


@functools.lru_cache(maxsize=4)
def _make_sc_main(bg):
    bpw_g = bg // _NW
    bpw_f = B // _NW

    @functools.partial(
        pl.kernel,
        mesh=plsc.VectorSubcoreMesh(**_MESH),
        out_type=(
            jax.ShapeDtypeStruct((bg, D), jnp.float32),
            jax.ShapeDtypeStruct((_F, B), jnp.float32),
        ),
        scratch_types=[
            pltpu.VMEM((bpw_g,), jnp.int32),
            pltpu.VMEM((bpw_f,), jnp.int32),
            pltpu.VMEM((bpw_f,), jnp.float32),
            pltpu.VMEM((2, bpw_f), jnp.float32),
            pltpu.VMEM((bpw_g,), jnp.int32),
            pltpu.VMEM((bpw_g, D), jnp.float32),
            pltpu.VMEM((_F, bpw_f), jnp.float32),
            pltpu.SemaphoreType.DMA,
            pltpu.SemaphoreType.DMA,
            pltpu.SemaphoreType.DMA,
            pltpu.SemaphoreType.DMA,
            pltpu.SemaphoreType.DMA,
        ],
    )
    def sc_main(emb_hbm, node_hbm, code_hbm, ct_hbm, twt_hbm,
                rows_hbm, featst_hbm, node_v, code_v, ct_v, tw_v, emb_idx_v,
                rows_v, featst_v,
                sem_a, sem_n, sem_c, sem_t, sem_w):
        wid = _wid()
        fbase = wid * bpw_f
        lane = lax.broadcasted_iota(jnp.int32, (16,), 0)

        cp_c = pltpu.async_copy(code_hbm.at[pl.ds(fbase, bpw_f)], code_v, sem_c)
        cp_t = pltpu.async_copy(ct_hbm.at[pl.ds(fbase, bpw_f)], ct_v, sem_t)
        cp_w = pltpu.async_copy(
            twt_hbm.at[0, :, pl.ds(fbase, bpw_f)], tw_v, sem_w)

        emb_cp = _gather_rows(emb_hbm, node_hbm, rows_hbm, node_v, emb_idx_v,
                              rows_v, sem_n, sem_a, wid * bpw_g, wid * bpw_g,
                              bpw_g, lane)

        cp_t.wait()
        cp_w.wait()
        for i in range(bpw_f // 16):
            sl = pl.ds(i * 16, 16)
            featst_v[0, sl] = ct_v[sl] / tw_v[1, sl]

        cp_c.wait()
        for i in range(bpw_f // 16):
            sl = pl.ds(i * 16, 16)
            code16 = code_v[sl]
            btf = (code16 & 1).astype(jnp.float32)
            inff = ((code16 >> 1) & 1).astype(jnp.float32)
            featst_v[6, sl] = 1.0 - btf
            featst_v[7, sl] = btf
            featst_v[8, sl] = 1.0 - inff
            featst_v[9, sl] = inff
            featst_v[10, sl] = (code16 >> 2).astype(jnp.float32)
        pltpu.sync_copy(featst_v, featst_hbm.at[:, pl.ds(fbase, bpw_f)])

        emb_cp.wait()
        pltpu.sync_copy(rows_v, rows_hbm.at[pl.ds(wid * bpw_g, bpw_g)])

    return sc_main


@functools.lru_cache(maxsize=4)
def _make_sc_tail(bc, off):
    bpw = bc // _NW

    @functools.partial(
        pl.kernel,
        mesh=plsc.VectorSubcoreMesh(**_MESH),
        out_type=jax.ShapeDtypeStruct((bc, D), jnp.float32),
        scratch_types=[
            pltpu.VMEM((bpw,), jnp.int32),
            pltpu.VMEM((bpw,), jnp.int32),
            pltpu.VMEM((bpw, D), jnp.float32),
            pltpu.SemaphoreType.DMA,
            pltpu.SemaphoreType.DMA,
        ],
    )
    def sc_tail(emb_hbm, node_hbm, rows_hbm, node_v, emb_idx_v, rows_v,
                sem_n, sem_a):
        wid = _wid()
        lane = lax.broadcasted_iota(jnp.int32, (16,), 0)
        cp = _gather_rows(emb_hbm, node_hbm, rows_hbm, node_v, emb_idx_v,
                          rows_v, sem_n, sem_a, off + wid * bpw, wid * bpw,
                          bpw, lane)
        cp.wait()
        pltpu.sync_copy(rows_v, rows_hbm.at[pl.ds(wid * bpw, bpw)])

    return sc_tail


_BLK = 2048


def _tc_body(x_ref, ft_ref, st_ref, w_ref, b_ref, out_ref):
    blk = x_ref.shape[0]
    w0 = w_ref[:D, :]
    w1 = w_ref[D:, :]
    sidx = ft_ref[10:11, :].astype(jnp.int32)
    srow = lax.broadcasted_iota(jnp.int32, (S, blk), 0)
    rc = jnp.sum(jnp.where(srow == sidx, st_ref[...], 0),
                 axis=0, keepdims=True)
    rc = jnp.clip(rc, 0, NUM_REV - 1)
    crow = lax.broadcasted_iota(jnp.int32, (NUM_REV, blk), 0)
    oh5 = jnp.where(crow == rc, 1.0, 0.0)
    ft = jnp.concatenate(
        [ft_ref[0:1, :], oh5, ft_ref[6:10, :]], axis=0)
    out_ref[...] = (
        jnp.dot(x_ref[...], w0, preferred_element_type=jnp.float32)
        + lax.dot_general(ft, w1, (((0,), (0,)), ((), ())),
                          preferred_element_type=jnp.float32)
        + b_ref[...][None, :]
    )


def _tc_body_alias(x_ref, ft_ref, st_ref, w_ref, b_ref, prev_ref, out_ref):
    _tc_body(x_ref, ft_ref, st_ref, w_ref, b_ref, out_ref)


@functools.lru_cache(maxsize=4)
def _make_tc_call(bc, off):
    blk_off = off // _BLK
    in_specs = [
        pl.BlockSpec((_BLK, D), lambda i: (i, 0)),
        pl.BlockSpec((_F, _BLK), lambda i: (0, i + blk_off)),
        pl.BlockSpec((S, _BLK), lambda i: (0, i + blk_off)),
        pl.BlockSpec((D + 10, D), lambda i: (0, 0)),
        pl.BlockSpec((D,), lambda i: (0,)),
    ]
    if off == 0:
        return pl.pallas_call(
            _tc_body,
            grid=(bc // _BLK,),
            in_specs=in_specs,
            out_specs=pl.BlockSpec((_BLK, D), lambda i: (i + blk_off, 0)),
            out_shape=jax.ShapeDtypeStruct((B, D), jnp.float32),
        )
    return pl.pallas_call(
        _tc_body_alias,
        grid=(bc // _BLK,),
        in_specs=in_specs + [pl.BlockSpec(memory_space=pltpu.MemorySpace.HBM)],
        out_specs=pl.BlockSpec((_BLK, D), lambda i: (i + blk_off, 0)),
        out_shape=jax.ShapeDtypeStruct((B, D), jnp.float32),
        input_output_aliases={5: 0},
    )


_NCHUNK = 1


def kernel(embeddings, current_node, revisit_count_stack, step_idx,
           backtrack_budget_reached, confirmed_infeasible,
           current_time, time_windows, W, b):
    emb2d = embeddings.reshape(B * N, D)
    code = (step_idx.astype(jnp.int32) * 4
            + backtrack_budget_reached.astype(jnp.int32)
            + 2 * confirmed_infeasible.astype(jnp.int32))
    stackt = revisit_count_stack.T.astype(jnp.int32)
    twt = time_windows.transpose(1, 2, 0)
    node = current_node.astype(jnp.int32)

    bc = B // _NCHUNK
    rows0, featst = _make_sc_main(bc)(
        emb2d, node, code, current_time, twt)
    out = _make_tc_call(bc, 0)(rows0, featst, stackt, W, b)
    for ci in range(1, _NCHUNK):
        rows = _make_sc_tail(bc, ci * bc)(emb2d, node)
        out = _make_tc_call(bc, ci * bc)(rows, featst, stackt, W, b, out)
    return out
